# baseline (speedup 1.0000x reference)
import functools

import jax
import jax.numpy as jnp
from jax import lax
from jax.experimental import pallas as pl
from jax.experimental.pallas import tpu as pltpu
from jax.experimental.pallas import tpu_sc as plsc

_N = 10000
_E = 160000
_D = 256
_H = 128
_ALPHA = 0.5

_NPAD = 10240
_TS = _NPAD // 16
_K = 64
_EPT = 10240
_E2 = 16 * _EPT
_ITERS = _EPT // _K

_mesh = plsc.VectorSubcoreMesh(core_axis_name="c", subcore_axis_name="s")


@functools.partial(
    pl.kernel,
    out_type=[
        jax.ShapeDtypeStruct((_NPAD,), jnp.float32),
        jax.ShapeDtypeStruct((_NPAD,), jnp.float32),
    ],
    mesh=_mesh,
    scratch_types=[
        pltpu.VMEM((_ITERS, _K), jnp.int32),
        pltpu.VMEM((_K,), jnp.float32),
        pltpu.VMEM_SHARED((_NPAD,), jnp.float32),
    ],
)
def _deg_kernel(row3_hbm, col3_hbm, zeros1_hbm, odeg_hbm, ideg_hbm,
                idx_v, ones_v, deg_sh):
    c = lax.axis_index("c")
    s = lax.axis_index("s")

    for j in range(_K // 16):
        ones_v[pl.ds(j * 16, 16)] = jnp.ones((16,), jnp.float32)

    pltpu.sync_copy(zeros1_hbm, deg_sh.at[pl.ds(s * _TS, _TS)])

    @pl.when(c == 0)
    def _():
        pltpu.sync_copy(row3_hbm.at[s], idx_v)

    @pl.when(c == 1)
    def _():
        pltpu.sync_copy(col3_hbm.at[s], idx_v)

    plsc.subcore_barrier()

    def body(i, carry):
        pltpu.sync_copy(ones_v, deg_sh.at[idx_v.at[i]], add=True)
        return carry
    lax.fori_loop(0, _ITERS, body, 0)
    plsc.subcore_barrier()

    @pl.when(c == 0)
    def _():
        pltpu.sync_copy(deg_sh.at[pl.ds(s * _TS, _TS)],
                        odeg_hbm.at[pl.ds(s * _TS, _TS)])

    @pl.when(c == 1)
    def _():
        pltpu.sync_copy(deg_sh.at[pl.ds(s * _TS, _TS)],
                        ideg_hbm.at[pl.ds(s * _TS, _TS)])


_NBUF = 5
_IBUF = 2 * _NBUF

@functools.partial(
    pl.kernel,
    out_type=[jax.ShapeDtypeStruct((_NPAD, _H), jnp.float32)] * 4,
    mesh=_mesh,
    scratch_types=(
        [pltpu.VMEM((_K,), jnp.int32)] * _IBUF
        + [pltpu.VMEM((_K,), jnp.int32)] * _IBUF
        + [pltpu.VMEM((_K, _H), jnp.float32)] * _NBUF
        + [pltpu.VMEM_SHARED((_NPAD, _H), jnp.float32)]
        + [pltpu.SemaphoreType.DMA] * _IBUF
        + [pltpu.SemaphoreType.DMA] * _NBUF
        + [pltpu.SemaphoreType.DMA]
    ),
)
def _edge_kernel(row_hbm, col_hbm, zsl_hbm, zsh_hbm, zdl_hbm, zdh_hbm,
                 zeros2_hbm, asl_hbm, ash_hbm, adl_hbm, adh_hbm,
                 *scratch):
    gi = scratch[:_IBUF]
    si = scratch[_IBUF:2 * _IBUF]
    rows = scratch[2 * _IBUF:2 * _IBUF + _NBUF]
    acc_sh = scratch[2 * _IBUF + _NBUF]
    isems = scratch[2 * _IBUF + _NBUF + 1:2 * _IBUF + _NBUF + 1 + _IBUF]
    gsems = scratch[2 * _IBUF + _NBUF + 1 + _IBUF:-1]
    ssem = scratch[-1]
    c = lax.axis_index("c")
    s = lax.axis_index("s")

    def do_pass(gather_hbm, scatter_hbm, ztab_hbm, out_hbm):
        pltpu.sync_copy(zeros2_hbm, acc_sh.at[pl.ds(s * _TS, _TS)])
        plsc.subcore_barrier()

        def issue_idx(chunk, j):
            base = pl.multiple_of(s * _EPT + chunk * _K, 8)
            pltpu.async_copy(gather_hbm.at[pl.ds(base, _K)], gi[j], isems[j])
            pltpu.async_copy(scatter_hbm.at[pl.ds(base, _K)], si[j], isems[j])

        def wait_idx(j):
            pltpu.make_async_copy(
                gather_hbm.at[pl.ds(0, _K)], gi[j], isems[j]).wait()
            pltpu.make_async_copy(
                scatter_hbm.at[pl.ds(0, _K)], si[j], isems[j]).wait()

        def wait_scatter(b, j):
            pltpu.make_async_copy(rows[b], acc_sh.at[si[j]], ssem).wait()

        for j in range(_IBUF):
            issue_idx(j, j)
        for b in range(_NBUF - 1):
            wait_idx(b)
            pltpu.async_copy(ztab_hbm.at[gi[b]], rows[b], gsems[b])

        def outer(t, carry):
            for bb in range(_IBUF):
                i = t * _IBUF + bb
                b = bb % _NBUF
                pltpu.make_async_copy(
                    ztab_hbm.at[gi[bb]], rows[b], gsems[b]).wait()
                bprev = (bb + _NBUF - 1) % _NBUF
                jprev = (bb + _IBUF - 1) % _IBUF

                @pl.when(i >= 1)
                def _():
                    wait_scatter(bprev, jprev)
                    inext2 = i - 1 + _IBUF

                    @pl.when(inext2 < _ITERS)
                    def _():
                        issue_idx(inext2, jprev)
                pltpu.async_copy(rows[b], acc_sh.at[si[bb]], ssem, add=True)

                inextg = i + _NBUF - 1
                jg = (bb + _NBUF - 1) % _IBUF

                @pl.when(inextg < _ITERS)
                def _():
                    wait_idx(jg)
                    pltpu.async_copy(ztab_hbm.at[gi[jg]], rows[bprev],
                                     gsems[bprev])
            return carry
        lax.fori_loop(0, _ITERS // _IBUF, outer, 0)
        wait_scatter((_ITERS - 1) % _NBUF, (_ITERS - 1) % _IBUF)
        plsc.subcore_barrier()
        pltpu.sync_copy(acc_sh.at[pl.ds(s * _TS, _TS)],
                        out_hbm.at[pl.ds(s * _TS, _TS)])
        plsc.subcore_barrier()

    @pl.when(c == 0)
    def _():
        do_pass(col_hbm, row_hbm, zsl_hbm, asl_hbm)
        do_pass(row_hbm, col_hbm, zdl_hbm, adl_hbm)

    @pl.when(c == 1)
    def _():
        do_pass(col_hbm, row_hbm, zsh_hbm, ash_hbm)
        do_pass(row_hbm, col_hbm, zdh_hbm, adh_hbm)


_RB = 1000


def _mm_body(x_ref, ws_ref, wd_ref, ys_ref, yd_ref):
    xb = x_ref[...]
    dn = (((1,), (1,)), ((), ()))
    ys_ref[...] = _ALPHA * lax.dot_general(
        xb, ws_ref[...], dn, preferred_element_type=jnp.float32)
    yd_ref[...] = (1.0 - _ALPHA) * lax.dot_general(
        xb, wd_ref[...], dn, preferred_element_type=jnp.float32)


_mm_call = pl.pallas_call(
    _mm_body,
    grid=(_N // _RB,),
    in_specs=[
        pl.BlockSpec((_RB, _D), lambda i: (i, 0)),
        pl.BlockSpec((_D, _D), lambda i: (0, 0)),
        pl.BlockSpec((_D, _D), lambda i: (0, 0)),
    ],
    out_specs=[pl.BlockSpec((_RB, _D), lambda i: (i, 0))] * 2,
    out_shape=[jax.ShapeDtypeStruct((_N, _D), jnp.float32)] * 2,
)


def _scale_body(ys_ref, yd_ref, od_ref, id_ref,
                zsl_ref, zsh_ref, zdl_ref, zdh_ref):
    od = od_ref[...]
    idg = id_ref[...]
    oinv = jnp.where(od > 0, lax.rsqrt(od), 0.0)
    iinv = jnp.where(idg > 0, lax.rsqrt(idg), 0.0)
    zs = iinv * ys_ref[...]
    zd = oinv * yd_ref[...]
    zsl_ref[...] = zs[:, :_H]
    zsh_ref[...] = zs[:, _H:]
    zdl_ref[...] = zd[:, :_H]
    zdh_ref[...] = zd[:, _H:]


_scale_call = pl.pallas_call(
    _scale_body,
    grid=(_N // _RB,),
    in_specs=[
        pl.BlockSpec((_RB, _D), lambda i: (i, 0)),
        pl.BlockSpec((_RB, _D), lambda i: (i, 0)),
        pl.BlockSpec((_RB, 1), lambda i: (i, 0)),
        pl.BlockSpec((_RB, 1), lambda i: (i, 0)),
    ],
    out_specs=[pl.BlockSpec((_RB, _H), lambda i: (i, 0))] * 4,
    out_shape=[jax.ShapeDtypeStruct((_NPAD, _H), jnp.float32)] * 4,
)


def _out_body(asl_ref, ash_ref, adl_ref, adh_ref, od_ref, id_ref,
              bs_ref, bd_ref, out_ref):
    od = od_ref[...]
    idg = id_ref[...]
    oinv = jnp.where(od > 0, lax.rsqrt(od), 0.0)
    iinv = jnp.where(idg > 0, lax.rsqrt(idg), 0.0)
    bias = _ALPHA * bs_ref[...] + (1.0 - _ALPHA) * bd_ref[...]
    out_ref[:, :_H] = oinv * asl_ref[...] + iinv * adl_ref[...] + bias[:, :_H]
    out_ref[:, _H:] = oinv * ash_ref[...] + iinv * adh_ref[...] + bias[:, _H:]


_out_call = pl.pallas_call(
    _out_body,
    grid=(_N // _RB,),
    in_specs=[
        pl.BlockSpec((_RB, _H), lambda i: (i, 0)),
        pl.BlockSpec((_RB, _H), lambda i: (i, 0)),
        pl.BlockSpec((_RB, _H), lambda i: (i, 0)),
        pl.BlockSpec((_RB, _H), lambda i: (i, 0)),
        pl.BlockSpec((_RB, 1), lambda i: (i, 0)),
        pl.BlockSpec((_RB, 1), lambda i: (i, 0)),
        pl.BlockSpec((1, _D), lambda i: (0, 0)),
        pl.BlockSpec((1, _D), lambda i: (0, 0)),
    ],
    out_specs=pl.BlockSpec((_RB, _D), lambda i: (i, 0)),
    out_shape=jax.ShapeDtypeStruct((_N, _D), jnp.float32),
)


def kernel(x, edge_index, W_src, b_src, W_dst, b_dst):
    pad = _N + (jnp.arange(_E2 - _E, dtype=jnp.int32) % (_NPAD - _N))
    row_p = jnp.concatenate([edge_index[0], pad])
    col_p = jnp.concatenate([edge_index[1], pad])
    row3 = row_p.reshape(16, _ITERS, _K)
    col3 = col_p.reshape(16, _ITERS, _K)
    zeros1 = jnp.zeros((_TS,), jnp.float32)
    zeros2 = jnp.zeros((_TS, _H), jnp.float32)

    odeg, ideg = _deg_kernel(row3, col3, zeros1)
    od2 = odeg[:, None]
    id2 = ideg[:, None]

    ys, yd = _mm_call(x, W_src, W_dst)
    zsl, zsh, zdl, zdh = _scale_call(ys, yd, od2, id2)

    asl, ash, adl, adh = _edge_kernel(row_p, col_p, zsl, zsh, zdl, zdh, zeros2)

    return _out_call(asl, ash, adl, adh, od2, id2,
                     b_src[None, :], b_dst[None, :])


# speedup vs baseline: 1.0141x; 1.0141x over previous
import functools

import jax
import jax.numpy as jnp
from jax import lax
from jax.experimental import pallas as pl
from jax.experimental.pallas import tpu as pltpu
from jax.experimental.pallas import tpu_sc as plsc

_N = 10000
_E = 160000
_D = 256
_H = 128
_ALPHA = 0.5

_NPAD = 10240
_TS = _NPAD // 16
_K = 80
_EPT = 10240
_E2 = 16 * _EPT
_ITERS = _EPT // _K

_mesh = plsc.VectorSubcoreMesh(core_axis_name="c", subcore_axis_name="s")


@functools.partial(
    pl.kernel,
    out_type=[
        jax.ShapeDtypeStruct((_NPAD,), jnp.float32),
        jax.ShapeDtypeStruct((_NPAD,), jnp.float32),
    ],
    mesh=_mesh,
    scratch_types=[
        pltpu.VMEM((_ITERS, _K), jnp.int32),
        pltpu.VMEM((_K,), jnp.float32),
        pltpu.VMEM_SHARED((_NPAD,), jnp.float32),
    ],
)
def _deg_kernel(row3_hbm, col3_hbm, zeros1_hbm, odeg_hbm, ideg_hbm,
                idx_v, ones_v, deg_sh):
    c = lax.axis_index("c")
    s = lax.axis_index("s")

    for j in range(_K // 16):
        ones_v[pl.ds(j * 16, 16)] = jnp.ones((16,), jnp.float32)

    pltpu.sync_copy(zeros1_hbm, deg_sh.at[pl.ds(s * _TS, _TS)])

    @pl.when(c == 0)
    def _():
        pltpu.sync_copy(row3_hbm.at[s], idx_v)

    @pl.when(c == 1)
    def _():
        pltpu.sync_copy(col3_hbm.at[s], idx_v)

    plsc.subcore_barrier()

    def body(i, carry):
        pltpu.sync_copy(ones_v, deg_sh.at[idx_v.at[i]], add=True)
        return carry
    lax.fori_loop(0, _ITERS, body, 0)
    plsc.subcore_barrier()

    @pl.when(c == 0)
    def _():
        pltpu.sync_copy(deg_sh.at[pl.ds(s * _TS, _TS)],
                        odeg_hbm.at[pl.ds(s * _TS, _TS)])

    @pl.when(c == 1)
    def _():
        pltpu.sync_copy(deg_sh.at[pl.ds(s * _TS, _TS)],
                        ideg_hbm.at[pl.ds(s * _TS, _TS)])


_NBUF = 4
_IBUF = 2 * _NBUF

@functools.partial(
    pl.kernel,
    out_type=[jax.ShapeDtypeStruct((_NPAD, _H), jnp.float32)] * 4,
    mesh=_mesh,
    scratch_types=(
        [pltpu.VMEM((_K,), jnp.int32)] * _IBUF
        + [pltpu.VMEM((_K,), jnp.int32)] * _IBUF
        + [pltpu.VMEM((_K, _H), jnp.float32)] * _NBUF
        + [pltpu.VMEM_SHARED((_NPAD, _H), jnp.float32)]
        + [pltpu.SemaphoreType.DMA] * _IBUF
        + [pltpu.SemaphoreType.DMA] * _NBUF
    ),
)
def _edge_kernel(row_hbm, col_hbm, zsl_hbm, zsh_hbm, zdl_hbm, zdh_hbm,
                 zeros2_hbm, asl_hbm, ash_hbm, adl_hbm, adh_hbm,
                 *scratch):
    gi = scratch[:_IBUF]
    si = scratch[_IBUF:2 * _IBUF]
    rows = scratch[2 * _IBUF:2 * _IBUF + _NBUF]
    acc_sh = scratch[2 * _IBUF + _NBUF]
    isems = scratch[2 * _IBUF + _NBUF + 1:2 * _IBUF + _NBUF + 1 + _IBUF]
    gsems = scratch[2 * _IBUF + _NBUF + 1 + _IBUF:]
    c = lax.axis_index("c")
    s = lax.axis_index("s")

    def do_pass(gather_hbm, scatter_hbm, ztab_hbm, out_hbm):
        pltpu.sync_copy(zeros2_hbm, acc_sh.at[pl.ds(s * _TS, _TS)])
        plsc.subcore_barrier()

        def issue_idx(chunk, j):
            base = pl.multiple_of(s * _EPT + chunk * _K, 8)
            pltpu.async_copy(gather_hbm.at[pl.ds(base, _K)], gi[j], isems[j])
            pltpu.async_copy(scatter_hbm.at[pl.ds(base, _K)], si[j], isems[j])

        def wait_idx(j):
            pltpu.make_async_copy(
                gather_hbm.at[pl.ds(0, _K)], gi[j], isems[j]).wait()
            pltpu.make_async_copy(
                scatter_hbm.at[pl.ds(0, _K)], si[j], isems[j]).wait()

        for j in range(_IBUF):
            issue_idx(j, j)
        for b in range(_NBUF):
            wait_idx(b)
            pltpu.async_copy(ztab_hbm.at[gi[b]], rows[b], gsems[b])

        def outer(t, carry):
            for bb in range(_IBUF):
                i = t * _IBUF + bb
                b = bb % _NBUF
                pltpu.make_async_copy(
                    ztab_hbm.at[gi[bb]], rows[b], gsems[b]).wait()
                pltpu.sync_copy(rows[b], acc_sh.at[si[bb]], add=True)
                inext2 = i + _IBUF

                @pl.when(inext2 < _ITERS)
                def _():
                    issue_idx(inext2, bb)

                inextg = i + _NBUF
                jg = (bb + _NBUF) % _IBUF

                @pl.when(inextg < _ITERS)
                def _():
                    wait_idx(jg)
                    pltpu.async_copy(ztab_hbm.at[gi[jg]], rows[b], gsems[b])
            return carry
        lax.fori_loop(0, _ITERS // _IBUF, outer, 0)
        plsc.subcore_barrier()
        pltpu.sync_copy(acc_sh.at[pl.ds(s * _TS, _TS)],
                        out_hbm.at[pl.ds(s * _TS, _TS)])
        plsc.subcore_barrier()

    @pl.when(c == 0)
    def _():
        do_pass(col_hbm, row_hbm, zsl_hbm, asl_hbm)
        do_pass(row_hbm, col_hbm, zdl_hbm, adl_hbm)

    @pl.when(c == 1)
    def _():
        do_pass(col_hbm, row_hbm, zsh_hbm, ash_hbm)
        do_pass(row_hbm, col_hbm, zdh_hbm, adh_hbm)


_RB = 1000


def _mm_body(x_ref, ws_ref, wd_ref, od_ref, id_ref,
             zsl_ref, zsh_ref, zdl_ref, zdh_ref):
    xb = x_ref[...]
    od = od_ref[...]
    idg = id_ref[...]
    oinv = jnp.where(od > 0, lax.rsqrt(od), 0.0)
    iinv = jnp.where(idg > 0, lax.rsqrt(idg), 0.0)
    dn = (((1,), (1,)), ((), ()))
    zs = (_ALPHA * iinv) * lax.dot_general(
        xb, ws_ref[...], dn, preferred_element_type=jnp.float32)
    zd = ((1.0 - _ALPHA) * oinv) * lax.dot_general(
        xb, wd_ref[...], dn, preferred_element_type=jnp.float32)
    zsl_ref[...] = zs[:, :_H]
    zsh_ref[...] = zs[:, _H:]
    zdl_ref[...] = zd[:, :_H]
    zdh_ref[...] = zd[:, _H:]


_mm_call = pl.pallas_call(
    _mm_body,
    grid=(_N // _RB,),
    in_specs=[
        pl.BlockSpec((_RB, _D), lambda i: (i, 0)),
        pl.BlockSpec((_D, _D), lambda i: (0, 0)),
        pl.BlockSpec((_D, _D), lambda i: (0, 0)),
        pl.BlockSpec((_RB, 1), lambda i: (i, 0)),
        pl.BlockSpec((_RB, 1), lambda i: (i, 0)),
    ],
    out_specs=[pl.BlockSpec((_RB, _H), lambda i: (i, 0))] * 4,
    out_shape=[jax.ShapeDtypeStruct((_NPAD, _H), jnp.float32)] * 4,
)


def _out_body(asl_ref, ash_ref, adl_ref, adh_ref, od_ref, id_ref,
              bs_ref, bd_ref, out_ref):
    od = od_ref[...]
    idg = id_ref[...]
    oinv = jnp.where(od > 0, lax.rsqrt(od), 0.0)
    iinv = jnp.where(idg > 0, lax.rsqrt(idg), 0.0)
    bias = _ALPHA * bs_ref[...] + (1.0 - _ALPHA) * bd_ref[...]
    out_ref[:, :_H] = oinv * asl_ref[...] + iinv * adl_ref[...] + bias[:, :_H]
    out_ref[:, _H:] = oinv * ash_ref[...] + iinv * adh_ref[...] + bias[:, _H:]


_out_call = pl.pallas_call(
    _out_body,
    grid=(_N // _RB,),
    in_specs=[
        pl.BlockSpec((_RB, _H), lambda i: (i, 0)),
        pl.BlockSpec((_RB, _H), lambda i: (i, 0)),
        pl.BlockSpec((_RB, _H), lambda i: (i, 0)),
        pl.BlockSpec((_RB, _H), lambda i: (i, 0)),
        pl.BlockSpec((_RB, 1), lambda i: (i, 0)),
        pl.BlockSpec((_RB, 1), lambda i: (i, 0)),
        pl.BlockSpec((1, _D), lambda i: (0, 0)),
        pl.BlockSpec((1, _D), lambda i: (0, 0)),
    ],
    out_specs=pl.BlockSpec((_RB, _D), lambda i: (i, 0)),
    out_shape=jax.ShapeDtypeStruct((_N, _D), jnp.float32),
)


def kernel(x, edge_index, W_src, b_src, W_dst, b_dst):
    pad = _N + (jnp.arange(_E2 - _E, dtype=jnp.int32) % (_NPAD - _N))
    row_p = jnp.concatenate([edge_index[0], pad])
    col_p = jnp.concatenate([edge_index[1], pad])
    row3 = row_p.reshape(16, _ITERS, _K)
    col3 = col_p.reshape(16, _ITERS, _K)
    zeros1 = jnp.zeros((_TS,), jnp.float32)
    zeros2 = jnp.zeros((_TS, _H), jnp.float32)

    odeg, ideg = _deg_kernel(row3, col3, zeros1)
    od2 = odeg[:, None]
    id2 = ideg[:, None]

    zsl, zsh, zdl, zdh = _mm_call(x, W_src, W_dst, od2, id2)

    asl, ash, adl, adh = _edge_kernel(row_p, col_p, zsl, zsh, zdl, zdh, zeros2)

    return _out_call(asl, ash, adl, adh, od2, id2,
                     b_src[None, :], b_dst[None, :])
